# Initial kernel scaffold; baseline (speedup 1.0000x reference)
import jax
import jax.numpy as jnp
from jax.experimental import pallas as pl


def kernel(features, w0, b0, w1, b1, w2, b2, nbr0, nbr1, nbr2):
    raise NotImplementedError("write your pallas kernel here")


# speedup vs baseline: 3.5965x; 3.5965x over previous
import functools

import jax
import jax.numpy as jnp
from jax.experimental import pallas as pl
from jax.experimental.pallas import tpu as pltpu

_FANOUT = 4


def _gather_sum_tile(idx_ref, src, buf, base, m, p):
    S = m + 1
    for mi in range(m):
        o = base + _FANOUT * mi
        acc = None
        for k in range(_FANOUT):
            ik = pl.multiple_of(idx_ref[o + k], p)
            slab = src[pl.ds(ik, p), :]
            acc = slab if acc is None else acc + slab
        buf[mi:mi + p * S:S, :] = acc
    return jnp.concatenate([buf[c * S:c * S + m, :] for c in range(p)],
                           axis=-1)


def _l0_kernel(idx_ref, feat_hbm, w_ref, b_ref, o_ref, fbuf, buf, sem,
               *, m, nt, p):
    j = pl.program_id(1)

    @pl.when(j == 0)
    def _copy_src():
        cp = pltpu.make_async_copy(feat_hbm, fbuf, sem)
        cp.start()
        cp.wait()

    t = pl.program_id(0) * nt + j
    x = _gather_sum_tile(idx_ref, fbuf, buf, t * (m * _FANOUT), m, p)
    y = jnp.dot(x, w_ref[...], preferred_element_type=jnp.float32) + b_ref[...]
    o_ref[...] = jnp.maximum(y, 0.0)


def _l1_kernel(idx_ref, src_ref, w1_ref, wa_ref, wb_ref, b1_ref, o_ref, buf,
               *, m, nt, p):
    t = pl.program_id(0) * nt + pl.program_id(1)
    x = _gather_sum_tile(idx_ref, src_ref, buf, t * (m * _FANOUT), m, p)
    y = jnp.dot(x, w1_ref[...], preferred_element_type=jnp.float32) + b1_ref[...]
    yr = jnp.maximum(y, 0.0)
    o_ref[...] = (jnp.dot(y, wa_ref[...], preferred_element_type=jnp.float32)
                  + jnp.dot(yr, wb_ref[...], preferred_element_type=jnp.float32))


def _l2_kernel(idx_ref, src_ref, b2_ref, o_ref, *, m, nt):
    t = pl.program_id(0) * nt + pl.program_id(1)
    base = t * (m * _FANOUT)
    bias = b2_ref[0]
    for mi in range(m):
        o = base + _FANOUT * mi
        acc = (src_ref[idx_ref[o], 0] + src_ref[idx_ref[o + 1], 0]
               + src_ref[idx_ref[o + 2], 0] + src_ref[idx_ref[o + 3], 0])
        o_ref[mi] = acc + bias


def _layer0(feat2, idx, w, b, *, m):
    n_dst = idx.shape[0] // _FANOUT
    fin, fout = w.shape
    p = fin // 128
    nt = n_dst // (2 * m)
    kern = functools.partial(_l0_kernel, m=m, nt=nt, p=p)
    return pl.pallas_call(
        kern,
        out_shape=jax.ShapeDtypeStruct((n_dst, fout), jnp.float32),
        grid_spec=pltpu.PrefetchScalarGridSpec(
            num_scalar_prefetch=1,
            grid=(2, nt),
            in_specs=[
                pl.BlockSpec(memory_space=pl.ANY),
                pl.BlockSpec((fin, fout), lambda i, j, idx: (0, 0)),
                pl.BlockSpec((1, fout), lambda i, j, idx: (0, 0)),
            ],
            out_specs=pl.BlockSpec((m, fout), lambda i, j, idx: (i * nt + j, 0)),
            scratch_shapes=[
                pltpu.VMEM(feat2.shape, jnp.float32),
                pltpu.VMEM((p * (m + 1), 128), jnp.float32),
                pltpu.SemaphoreType.DMA,
            ],
        ),
        compiler_params=pltpu.CompilerParams(
            dimension_semantics=("parallel", "arbitrary"),
            vmem_limit_bytes=48 << 20,
        ),
    )(idx, feat2, w, b)


def _layer1(src2, idx, w1, wa, wb, b1, *, m):
    n_dst = idx.shape[0] // _FANOUT
    fin, fmid = w1.shape
    fout = wa.shape[1]
    p = fin // 128
    nt = n_dst // (2 * m)
    kern = functools.partial(_l1_kernel, m=m, nt=nt, p=p)
    return pl.pallas_call(
        kern,
        out_shape=jax.ShapeDtypeStruct((n_dst, fout), jnp.float32),
        grid_spec=pltpu.PrefetchScalarGridSpec(
            num_scalar_prefetch=1,
            grid=(2, nt),
            in_specs=[
                pl.BlockSpec(src2.shape, lambda i, j, idx: (0, 0)),
                pl.BlockSpec((fin, fmid), lambda i, j, idx: (0, 0)),
                pl.BlockSpec((fmid, fout), lambda i, j, idx: (0, 0)),
                pl.BlockSpec((fmid, fout), lambda i, j, idx: (0, 0)),
                pl.BlockSpec((1, fmid), lambda i, j, idx: (0, 0)),
            ],
            out_specs=pl.BlockSpec((m, fout), lambda i, j, idx: (i * nt + j, 0)),
            scratch_shapes=[
                pltpu.VMEM((p * (m + 1), 128), jnp.float32),
            ],
        ),
        compiler_params=pltpu.CompilerParams(
            dimension_semantics=("parallel", "arbitrary"),
            vmem_limit_bytes=40 << 20,
        ),
    )(idx, src2, w1, wa, wb, b1)


def _layer2(src3, idx, b2, *, m):
    n_dst = idx.shape[0] // _FANOUT
    fout = src3.shape[-1]
    nt = n_dst // (2 * m)
    kern = functools.partial(_l2_kernel, m=m, nt=nt)
    return pl.pallas_call(
        kern,
        out_shape=jax.ShapeDtypeStruct((n_dst, fout), jnp.float32),
        grid_spec=pltpu.PrefetchScalarGridSpec(
            num_scalar_prefetch=1,
            grid=(2, nt),
            in_specs=[
                pl.BlockSpec(src3.shape, lambda i, j, idx: (0, 0, 0)),
                pl.BlockSpec((1, fout), lambda i, j, idx: (0, 0)),
            ],
            out_specs=pl.BlockSpec((m, fout), lambda i, j, idx: (i * nt + j, 0)),
            scratch_shapes=[],
        ),
        compiler_params=pltpu.CompilerParams(
            dimension_semantics=("parallel", "arbitrary"),
            vmem_limit_bytes=16 << 20,
        ),
    )(idx, src3, b2)


def kernel(features, w0, b0, w1, b1, w2, b2, nbr0, nbr1, nbr2):
    f32 = jnp.float32
    fin = features.shape[1]
    fmid = w1.shape[0]

    feat2 = features.astype(f32).reshape(-1, 128)
    idx0 = (nbr0.astype(jnp.int32) * (fin // 128)).reshape(-1)
    w0s = (w0.astype(f32) / _FANOUT)
    h1 = _layer0(feat2, idx0, w0s, b0.astype(f32).reshape(1, -1), m=128)

    idx1 = (nbr1.astype(jnp.int32) * (fmid // 128)).reshape(-1)
    w1s = (w1.astype(f32) / _FANOUT)
    wa = (w2[:fmid].astype(f32) / _FANOUT)
    wb = (w2[fmid:].astype(f32) / _FANOUT)
    z = _layer1(h1.reshape(-1, 128), idx1, w1s, wa, wb,
                b1.astype(f32).reshape(1, -1), m=128)

    idx2 = nbr2.astype(jnp.int32).reshape(-1)
    out = _layer2(z.reshape(z.shape[0], 1, z.shape[1]), idx2,
                  b2.astype(f32).reshape(1, -1), m=128)
    return out.astype(f32)


# speedup vs baseline: 4.1499x; 1.1539x over previous
import functools

import jax
import jax.numpy as jnp
from jax.experimental import pallas as pl
from jax.experimental.pallas import tpu as pltpu

_FANOUT = 4


def _gather_sum_tile(idx_ref, src, buf, base, m, p):
    S = m + 1
    for mi in range(m):
        o = base + _FANOUT * mi
        acc = None
        for k in range(_FANOUT):
            ik = pl.multiple_of(idx_ref[o + k], p)
            slab = src[pl.ds(ik, p), :]
            acc = slab if acc is None else acc + slab
        buf[mi:mi + p * S:S, :] = acc
    return jnp.concatenate([buf[c * S:c * S + m, :] for c in range(p)],
                           axis=-1)


def _l0_kernel(idx_ref, feat_hbm, w_ref, b_ref, o_ref, fbuf, buf, sem,
               *, m, nt, p):
    j = pl.program_id(1)

    @pl.when(j == 0)
    def _copy_src():
        cp = pltpu.make_async_copy(feat_hbm, fbuf, sem)
        cp.start()
        cp.wait()

    t = pl.program_id(0) * nt + j
    x = _gather_sum_tile(idx_ref, fbuf, buf, t * (m * _FANOUT), m, p)
    y = jnp.dot(x, w_ref[...], preferred_element_type=jnp.float32) + b_ref[...]
    h = jnp.maximum(y, 0.0)
    o_ref[0:2 * m:2, :] = h[:, :128]
    o_ref[1:2 * m:2, :] = h[:, 128:]


def _l1_kernel(idx_ref, src_ref, w1_ref, wa_ref, wb_ref, b1_ref, o_ref, buf,
               *, m, nt, p):
    t = pl.program_id(0) * nt + pl.program_id(1)
    x = _gather_sum_tile(idx_ref, src_ref, buf, t * (m * _FANOUT), m, p)
    y = jnp.dot(x, w1_ref[...], preferred_element_type=jnp.float32) + b1_ref[...]
    yr = jnp.maximum(y, 0.0)
    o_ref[...] = (jnp.dot(y, wa_ref[...], preferred_element_type=jnp.float32)
                  + jnp.dot(yr, wb_ref[...], preferred_element_type=jnp.float32))


def _l2_kernel(idx_ref, src_ref, b2_ref, o_ref, *, m, nt):
    t = pl.program_id(0) * nt + pl.program_id(1)
    base = t * (m * _FANOUT)
    bias = b2_ref[0]
    for mi in range(m):
        o = base + _FANOUT * mi
        acc = (src_ref[idx_ref[o], 0] + src_ref[idx_ref[o + 1], 0]
               + src_ref[idx_ref[o + 2], 0] + src_ref[idx_ref[o + 3], 0])
        o_ref[mi] = acc + bias


def _layer0(feat2, idx, w, b, *, m):
    n_dst = idx.shape[0] // _FANOUT
    fin, fout = w.shape
    p = fin // 128
    nt = n_dst // (2 * m)
    kern = functools.partial(_l0_kernel, m=m, nt=nt, p=p)
    return pl.pallas_call(
        kern,
        out_shape=jax.ShapeDtypeStruct((n_dst * (fout // 128), 128), jnp.float32),
        grid_spec=pltpu.PrefetchScalarGridSpec(
            num_scalar_prefetch=1,
            grid=(2, nt),
            in_specs=[
                pl.BlockSpec(memory_space=pl.ANY),
                pl.BlockSpec((fin, fout), lambda i, j, idx: (0, 0)),
                pl.BlockSpec((1, fout), lambda i, j, idx: (0, 0)),
            ],
            out_specs=pl.BlockSpec((m * (fout // 128), 128),
                                   lambda i, j, idx: (i * nt + j, 0)),
            scratch_shapes=[
                pltpu.VMEM(feat2.shape, jnp.float32),
                pltpu.VMEM((p * (m + 1), 128), jnp.float32),
                pltpu.SemaphoreType.DMA,
            ],
        ),
        compiler_params=pltpu.CompilerParams(
            dimension_semantics=("parallel", "arbitrary"),
            vmem_limit_bytes=48 << 20,
        ),
    )(idx, feat2, w, b)


def _layer1(src2, idx, w1, wa, wb, b1, *, m):
    n_dst = idx.shape[0] // _FANOUT
    fin, fmid = w1.shape
    fout = wa.shape[1]
    p = fin // 128
    nt = n_dst // (2 * m)
    kern = functools.partial(_l1_kernel, m=m, nt=nt, p=p)
    return pl.pallas_call(
        kern,
        out_shape=jax.ShapeDtypeStruct((n_dst, fout), jnp.float32),
        grid_spec=pltpu.PrefetchScalarGridSpec(
            num_scalar_prefetch=1,
            grid=(2, nt),
            in_specs=[
                pl.BlockSpec(src2.shape, lambda i, j, idx: (0, 0)),
                pl.BlockSpec((fin, fmid), lambda i, j, idx: (0, 0)),
                pl.BlockSpec((fmid, fout), lambda i, j, idx: (0, 0)),
                pl.BlockSpec((fmid, fout), lambda i, j, idx: (0, 0)),
                pl.BlockSpec((1, fmid), lambda i, j, idx: (0, 0)),
            ],
            out_specs=pl.BlockSpec((m, fout), lambda i, j, idx: (i * nt + j, 0)),
            scratch_shapes=[
                pltpu.VMEM((p * (m + 1), 128), jnp.float32),
            ],
        ),
        compiler_params=pltpu.CompilerParams(
            dimension_semantics=("parallel", "arbitrary"),
            vmem_limit_bytes=40 << 20,
        ),
    )(idx, src2, w1, wa, wb, b1)


def _layer2(src3, idx, b2, *, m):
    n_dst = idx.shape[0] // _FANOUT
    fout = src3.shape[-1]
    nt = n_dst // (2 * m)
    kern = functools.partial(_l2_kernel, m=m, nt=nt)
    return pl.pallas_call(
        kern,
        out_shape=jax.ShapeDtypeStruct((n_dst, fout), jnp.float32),
        grid_spec=pltpu.PrefetchScalarGridSpec(
            num_scalar_prefetch=1,
            grid=(2, nt),
            in_specs=[
                pl.BlockSpec(src3.shape, lambda i, j, idx: (0, 0, 0)),
                pl.BlockSpec((1, fout), lambda i, j, idx: (0, 0)),
            ],
            out_specs=pl.BlockSpec((m, fout), lambda i, j, idx: (i * nt + j, 0)),
            scratch_shapes=[],
        ),
        compiler_params=pltpu.CompilerParams(
            dimension_semantics=("parallel", "arbitrary"),
            vmem_limit_bytes=16 << 20,
        ),
    )(idx, src3, b2)


def kernel(features, w0, b0, w1, b1, w2, b2, nbr0, nbr1, nbr2):
    f32 = jnp.float32
    fin = features.shape[1]
    fmid = w1.shape[0]

    feat2 = features.astype(f32).reshape(-1, 128)
    idx0 = (nbr0.astype(jnp.int32) * (fin // 128)).reshape(-1)
    w0s = (w0.astype(f32) / _FANOUT)
    h1v = _layer0(feat2, idx0, w0s, b0.astype(f32).reshape(1, -1), m=256)

    idx1 = (nbr1.astype(jnp.int32) * (fmid // 128)).reshape(-1)
    w1s = (w1.astype(f32) / _FANOUT)
    wa = (w2[:fmid].astype(f32) / _FANOUT)
    wb = (w2[fmid:].astype(f32) / _FANOUT)
    z = _layer1(h1v, idx1, w1s, wa, wb,
                b1.astype(f32).reshape(1, -1), m=256)

    idx2 = nbr2.astype(jnp.int32).reshape(-1)
    out = _layer2(z.reshape(z.shape[0], 1, z.shape[1]), idx2,
                  b2.astype(f32).reshape(1, -1), m=256)
    return out.astype(f32)
